# baseline (speedup 1.0000x reference)
import jax
import jax.numpy as jnp
from jax import lax
from jax.experimental import pallas as pl
from jax.experimental.pallas import tpu as pltpu, tpu_sc as plsc

N = 10000
E = 320000
D = 128
H = 128

DA = 144
CHUNK = 128
NUM_TILES = 16
CHUNKS_PER_TILE = 160
CHUNKS_PER_BRANCH = NUM_TILES * CHUNKS_PER_TILE
E_PAD = CHUNKS_PER_BRANCH * CHUNK - E
DUMMY_DST = N
ROWS_MAIN = 624
TAIL_ROWS = N - NUM_TILES * ROWS_MAIN
N_PAD = 10112
ZPT = 632
BLK = 8
NBLK = CHUNKS_PER_TILE // BLK


def _sc_body(x_hbm, src_hbm, dst_hbm, agg_out,
             src_blk, dst_blk, rows_a, rows_b,
             agg_sh, gsem_a, gsem_b):
    c = lax.axis_index("c")
    s = lax.axis_index("s")

    zero16 = jnp.zeros((16,), jnp.float32)

    def fill_rows_zero(i, carry):
        for k in range(DA // 16):
            rows_a[i, pl.ds(16 * k, 16)] = zero16
            rows_b[i, pl.ds(16 * k, 16)] = zero16
        return carry

    lax.fori_loop(0, CHUNK, fill_rows_zero, 0)

    for i in range(4):
        pltpu.sync_copy(rows_a, agg_sh.at[pl.ds(s * ZPT + i * CHUNK, CHUNK)])
    pltpu.sync_copy(rows_b.at[pl.ds(0, ZPT - 4 * CHUNK)],
                    agg_sh.at[pl.ds(s * ZPT + 4 * CHUNK, ZPT - 4 * CHUNK)])

    plsc.subcore_barrier()

    tile_start = (c * NUM_TILES + s) * CHUNKS_PER_TILE

    def gather(buf, idx_row, sem):
        return pltpu.async_copy(x_hbm.at[src_blk.at[idx_row]], buf, sem)

    def gather_wait(buf, idx_row, sem):
        pltpu.make_async_copy(x_hbm.at[src_blk.at[idx_row]], buf, sem).wait()

    def scatter(buf, idx_row):
        pltpu.sync_copy(buf, agg_sh.at[dst_blk.at[idx_row]], add=True)

    pltpu.sync_copy(src_hbm.at[pl.ds(tile_start, BLK)], src_blk)
    pltpu.sync_copy(dst_hbm.at[pl.ds(tile_start, BLK)], dst_blk)
    gather(rows_a, 0, gsem_a)

    def block_body(b, carry):
        for p in range(BLK // 2):
            j0, j1 = 2 * p, 2 * p + 1
            gather_wait(rows_a, j0, gsem_a)
            gather(rows_b, j1, gsem_b)
            scatter(rows_a, j0)
            gather_wait(rows_b, j1, gsem_b)
            if p < BLK // 2 - 1:
                gather(rows_a, j0 + 2, gsem_a)
                scatter(rows_b, j1)
            else:
                @pl.when(b < NBLK - 1)
                def _refill():
                    pltpu.sync_copy(
                        src_hbm.at[pl.ds(tile_start + (b + 1) * BLK, BLK)],
                        src_blk)
                    gather(rows_a, 0, gsem_a)

                scatter(rows_b, j1)

        @pl.when(b < NBLK - 1)
        def _restage_dst():
            pltpu.sync_copy(
                dst_hbm.at[pl.ds(tile_start + (b + 1) * BLK, BLK)], dst_blk)

        return carry

    lax.fori_loop(0, NBLK, block_body, 0)

    plsc.subcore_barrier()

    base = s * ROWS_MAIN
    pltpu.sync_copy(agg_sh.at[pl.ds(base, ROWS_MAIN)],
                    agg_out.at[pl.ds(c * N + base, ROWS_MAIN)])

    @pl.when(s == NUM_TILES - 1)
    def _tail():
        tbase = NUM_TILES * ROWS_MAIN
        pltpu.sync_copy(agg_sh.at[pl.ds(tbase, TAIL_ROWS)],
                        agg_out.at[pl.ds(c * N + tbase, TAIL_ROWS)])


@jax.jit
def _sc_aggregate(x_aug, src2d, dst2d):
    mesh = plsc.VectorSubcoreMesh(core_axis_name="c", subcore_axis_name="s")
    return pl.kernel(
        _sc_body,
        out_type=jax.ShapeDtypeStruct((2 * N, DA), jnp.float32),
        mesh=mesh,
        compiler_params=pltpu.CompilerParams(use_tc_tiling_on_sc=False),
        scratch_types=[
            pltpu.VMEM((BLK, CHUNK), jnp.int32),
            pltpu.VMEM((BLK, CHUNK), jnp.int32),
            pltpu.VMEM((CHUNK, DA), jnp.float32),
            pltpu.VMEM((CHUNK, DA), jnp.float32),
            pltpu.VMEM_SHARED((N_PAD, DA), jnp.float32),
            pltpu.SemaphoreType.DMA,
            pltpu.SemaphoreType.DMA,
        ],
    )(x_aug, src2d, dst2d)


def _tc_body(agg_ref, w_ref, out_ref):
    w = w_ref[...]
    embs = []
    for c in range(2):
        a = agg_ref[c * N:(c + 1) * N, 0:D]
        deg = agg_ref[c * N:(c + 1) * N, D:D + 1]
        a = a / jnp.maximum(deg, 1.0)
        h = jnp.maximum(
            jax.lax.dot(a, w, preferred_element_type=jnp.float32), 0.0)
        embs.append(jnp.sum(h, axis=0, keepdims=True) / float(N))
    out_ref[...] = jnp.sum(embs[0] * embs[1]).reshape(1, 1)


@jax.jit
def _tc_finish(agg, W):
    return pl.pallas_call(
        _tc_body,
        out_shape=jax.ShapeDtypeStruct((1, 1), jnp.float32),
    )(agg, W)


def kernel(x1, x2, W, edge_index1, edge_index2):
    x_aug = jnp.concatenate(
        [jnp.concatenate([x1, x2], axis=0),
         jnp.ones((2 * N, DA - D), jnp.float32)], axis=1)
    src_pad = jnp.zeros((E_PAD,), jnp.int32)
    dst_pad = jnp.full((E_PAD,), DUMMY_DST, jnp.int32)
    src2d = jnp.concatenate(
        [edge_index1[0], src_pad, edge_index2[0] + N, src_pad]).reshape(-1, CHUNK)
    dst2d = jnp.concatenate(
        [edge_index1[1], dst_pad, edge_index2[1], dst_pad]).reshape(-1, CHUNK)
    agg = _sc_aggregate(x_aug, src2d, dst2d)
    out = _tc_finish(agg, W)
    return out[0, 0]


# speedup vs baseline: 2.4677x; 2.4677x over previous
import jax
import jax.numpy as jnp
from jax import lax
from jax.experimental import pallas as pl
from jax.experimental.pallas import tpu as pltpu, tpu_sc as plsc

N = 10000
E = 320000
D = 128
H = 128

DA = 144
CHUNK = 128
NUM_TILES = 16
CHUNKS_PER_TILE = 160
CHUNKS_PER_BRANCH = NUM_TILES * CHUNKS_PER_TILE
E_PAD = CHUNKS_PER_BRANCH * CHUNK - E
DUMMY_DST = N
ROWS_MAIN = 624
TAIL_ROWS = N - NUM_TILES * ROWS_MAIN
N_PAD = 10112
ZPT = 632
BLK = 8
NBLK = CHUNKS_PER_TILE // BLK


def _sc_body(x_hbm, src_hbm, dst_hbm, agg_out,
             src_blk, dst_blk, rows_a, rows_b,
             agg_sh, gsem_a, gsem_b):
    c = lax.axis_index("c")
    s = lax.axis_index("s")

    zero16 = jnp.zeros((16,), jnp.float32)

    def fill_rows_zero(i, carry):
        for k in range(DA // 16):
            rows_a[i, pl.ds(16 * k, 16)] = zero16
            rows_b[i, pl.ds(16 * k, 16)] = zero16
        return carry

    lax.fori_loop(0, CHUNK, fill_rows_zero, 0)

    for i in range(4):
        pltpu.sync_copy(rows_a, agg_sh.at[pl.ds(s * ZPT + i * CHUNK, CHUNK)])
    pltpu.sync_copy(rows_b.at[pl.ds(0, ZPT - 4 * CHUNK)],
                    agg_sh.at[pl.ds(s * ZPT + 4 * CHUNK, ZPT - 4 * CHUNK)])

    plsc.subcore_barrier()

    tile_start = (c * NUM_TILES + s) * CHUNKS_PER_TILE

    def gather(buf, idx_row, sem):
        return pltpu.async_copy(x_hbm.at[src_blk.at[idx_row]], buf, sem)

    def gather_wait(buf, idx_row, sem):
        pltpu.make_async_copy(x_hbm.at[src_blk.at[idx_row]], buf, sem).wait()

    def scatter(buf, idx_row):
        pltpu.sync_copy(buf, agg_sh.at[dst_blk.at[idx_row]], add=True)

    pltpu.sync_copy(src_hbm.at[pl.ds(tile_start, BLK)], src_blk)
    pltpu.sync_copy(dst_hbm.at[pl.ds(tile_start, BLK)], dst_blk)
    gather(rows_a, 0, gsem_a)

    def block_body(b, carry):
        for p in range(BLK // 2):
            j0, j1 = 2 * p, 2 * p + 1
            gather_wait(rows_a, j0, gsem_a)
            gather(rows_b, j1, gsem_b)
            scatter(rows_a, j0)
            gather_wait(rows_b, j1, gsem_b)
            if p < BLK // 2 - 1:
                gather(rows_a, j0 + 2, gsem_a)
                scatter(rows_b, j1)
            else:
                @pl.when(b < NBLK - 1)
                def _refill():
                    pltpu.sync_copy(
                        src_hbm.at[pl.ds(tile_start + (b + 1) * BLK, BLK)],
                        src_blk)
                    gather(rows_a, 0, gsem_a)

                scatter(rows_b, j1)

        @pl.when(b < NBLK - 1)
        def _restage_dst():
            pltpu.sync_copy(
                dst_hbm.at[pl.ds(tile_start + (b + 1) * BLK, BLK)], dst_blk)

        return carry

    lax.fori_loop(0, NBLK, block_body, 0)

    plsc.subcore_barrier()

    base = s * ROWS_MAIN
    pltpu.sync_copy(agg_sh.at[pl.ds(base, ROWS_MAIN)],
                    agg_out.at[pl.ds(c * N + base, ROWS_MAIN)])

    @pl.when(s == NUM_TILES - 1)
    def _tail():
        tbase = NUM_TILES * ROWS_MAIN
        pltpu.sync_copy(agg_sh.at[pl.ds(tbase, TAIL_ROWS)],
                        agg_out.at[pl.ds(c * N + tbase, TAIL_ROWS)])


@jax.jit
def _sc_aggregate(x_aug, src2d, dst2d):
    mesh = plsc.VectorSubcoreMesh(core_axis_name="c", subcore_axis_name="s")
    return pl.kernel(
        _sc_body,
        out_type=jax.ShapeDtypeStruct((2 * N, DA), jnp.float32),
        mesh=mesh,
        compiler_params=pltpu.CompilerParams(use_tc_tiling_on_sc=False),
        scratch_types=[
            pltpu.VMEM((BLK, CHUNK), jnp.int32),
            pltpu.VMEM((BLK, CHUNK), jnp.int32),
            pltpu.VMEM((CHUNK, DA), jnp.float32),
            pltpu.VMEM((CHUNK, DA), jnp.float32),
            pltpu.VMEM_SHARED((N_PAD, DA), jnp.float32),
            pltpu.SemaphoreType.DMA,
            pltpu.SemaphoreType.DMA,
        ],
    )(x_aug, src2d, dst2d)


def _tc_body(agg_ref, w_ref, out_ref):
    w = w_ref[...]
    embs = []
    for c in range(2):
        a = agg_ref[c * N:(c + 1) * N, 0:D]
        deg = agg_ref[c * N:(c + 1) * N, D:D + 1]
        a = a / jnp.maximum(deg, 1.0)
        h = jnp.maximum(
            jax.lax.dot(a, w, preferred_element_type=jnp.float32), 0.0)
        embs.append(jnp.sum(h, axis=0, keepdims=True) / float(N))
    out_ref[...] = jnp.sum(embs[0] * embs[1]).reshape(1, 1)


@jax.jit
def _tc_finish(agg, W):
    return pl.pallas_call(
        _tc_body,
        out_shape=jax.ShapeDtypeStruct((1, 1), jnp.float32),
    )(agg, W)


def kernel(x1, x2, W, edge_index1, edge_index2):
    x_aug = jnp.concatenate(
        [jnp.concatenate([x1, x2], axis=0),
         jnp.ones((2 * N, DA - D), jnp.float32)], axis=1)
    pad_iota = jnp.arange(E_PAD, dtype=jnp.int32)
    src_pad = (pad_iota * 131) % N
    dst_pad = DUMMY_DST + pad_iota % (N_PAD - N)
    src2d = jnp.concatenate(
        [edge_index1[0], src_pad, edge_index2[0] + N, src_pad + N]).reshape(-1, CHUNK)
    dst2d = jnp.concatenate(
        [edge_index1[1], dst_pad, edge_index2[1], dst_pad]).reshape(-1, CHUNK)
    agg = _sc_aggregate(x_aug, src2d, dst2d)
    out = _tc_finish(agg, W)
    return out[0, 0]


# speedup vs baseline: 2.8901x; 1.1712x over previous
import jax
import jax.numpy as jnp
from jax import lax
from jax.experimental import pallas as pl
from jax.experimental.pallas import tpu as pltpu, tpu_sc as plsc

N = 10000
E = 320000
D = 128
H = 128

CHUNK = 128
NCHUNKS = E // CHUNK
NUM_TILES = 16
CNTW = 16
BLK = 4
NBLK = 156 // BLK
ROWS_PT = N // NUM_TILES


def _sc_body(x1_hbm, x2_hbm, s1_hbm, d1_hbm, s2_hbm, d2_hbm,
             agg_out, cnt_out,
             src_blk, dst_blk, rows_a, rows_b, ones_b, zr_b,
             agg_sh, cnt_sh, gsem_a, gsem_b):
    c = lax.axis_index("c")
    s = lax.axis_index("s")

    zero16 = jnp.zeros((16,), jnp.float32)
    one16 = jnp.ones((16,), jnp.float32)

    def fill_bufs(i, carry):
        for k in range(D // 16):
            rows_a[i, pl.ds(16 * k, 16)] = zero16
            rows_b[i, pl.ds(16 * k, 16)] = zero16
        ones_b[i, pl.ds(0, 16)] = one16
        zr_b[i, pl.ds(0, 16)] = zero16
        return carry

    lax.fori_loop(0, CHUNK, fill_bufs, 0)

    for i in range(4):
        pltpu.sync_copy(rows_a, agg_sh.at[pl.ds(s * ROWS_PT + i * CHUNK, CHUNK)])
        pltpu.sync_copy(zr_b, cnt_sh.at[pl.ds(s * ROWS_PT + i * CHUNK, CHUNK)])
    rem = ROWS_PT - 4 * CHUNK
    pltpu.sync_copy(rows_b.at[pl.ds(0, rem)],
                    agg_sh.at[pl.ds(s * ROWS_PT + 4 * CHUNK, rem)])
    pltpu.sync_copy(zr_b.at[pl.ds(0, rem)],
                    cnt_sh.at[pl.ds(s * ROWS_PT + 4 * CHUNK, rem)])

    plsc.subcore_barrier()

    base = s * NCHUNKS // NUM_TILES
    count = (s + 1) * NCHUNKS // NUM_TILES - base

    def run_branch(x_hbm, src_hbm, dst_hbm):
        def gather(buf, idx_row, sem):
            return pltpu.async_copy(x_hbm.at[src_blk.at[idx_row]], buf, sem)

        def gather_wait(buf, idx_row, sem):
            pltpu.make_async_copy(x_hbm.at[src_blk.at[idx_row]], buf, sem).wait()

        def scatter(buf, idx_row):
            pltpu.sync_copy(buf, agg_sh.at[dst_blk.at[idx_row]], add=True)
            pltpu.sync_copy(ones_b, cnt_sh.at[dst_blk.at[idx_row]], add=True)

        pltpu.sync_copy(src_hbm.at[pl.ds(base, BLK)], src_blk)
        pltpu.sync_copy(dst_hbm.at[pl.ds(base, BLK)], dst_blk)
        gather(rows_a, 0, gsem_a)

        def block_body(b, carry):
            for p in range(BLK // 2):
                j0, j1 = 2 * p, 2 * p + 1
                gather_wait(rows_a, j0, gsem_a)
                gather(rows_b, j1, gsem_b)
                scatter(rows_a, j0)
                gather_wait(rows_b, j1, gsem_b)
                if p < BLK // 2 - 1:
                    gather(rows_a, j0 + 2, gsem_a)
                    scatter(rows_b, j1)
                else:
                    @pl.when(b < NBLK - 1)
                    def _refill():
                        pltpu.sync_copy(
                            src_hbm.at[pl.ds(base + (b + 1) * BLK, BLK)],
                            src_blk)
                        gather(rows_a, 0, gsem_a)

                    scatter(rows_b, j1)

            @pl.when(b < NBLK - 1)
            def _restage_dst():
                pltpu.sync_copy(
                    dst_hbm.at[pl.ds(base + (b + 1) * BLK, BLK)], dst_blk)

            return carry

        lax.fori_loop(0, NBLK, block_body, 0)

        @pl.when(count == 157)
        def _extra():
            pltpu.sync_copy(src_hbm.at[pl.ds(base + 156, 1)],
                            src_blk.at[pl.ds(0, 1)])
            pltpu.sync_copy(dst_hbm.at[pl.ds(base + 156, 1)],
                            dst_blk.at[pl.ds(0, 1)])
            gather(rows_a, 0, gsem_a)
            gather_wait(rows_a, 0, gsem_a)
            scatter(rows_a, 0)

    @pl.when(c == 0)
    def _branch0():
        run_branch(x1_hbm, s1_hbm, d1_hbm)

    @pl.when(c == 1)
    def _branch1():
        run_branch(x2_hbm, s2_hbm, d2_hbm)

    plsc.subcore_barrier()

    rbase = s * ROWS_PT
    pltpu.sync_copy(agg_sh.at[pl.ds(rbase, ROWS_PT)],
                    agg_out.at[pl.ds(c * N + rbase, ROWS_PT)])
    pltpu.sync_copy(cnt_sh.at[pl.ds(rbase, ROWS_PT)],
                    cnt_out.at[pl.ds(c * N + rbase, ROWS_PT)])


@jax.jit
def _sc_aggregate(x1, x2, s1, d1, s2, d2):
    mesh = plsc.VectorSubcoreMesh(core_axis_name="c", subcore_axis_name="s")
    return pl.kernel(
        _sc_body,
        out_type=[jax.ShapeDtypeStruct((2 * N, D), jnp.float32),
                  jax.ShapeDtypeStruct((2 * N, CNTW), jnp.float32)],
        mesh=mesh,
        compiler_params=pltpu.CompilerParams(use_tc_tiling_on_sc=False),
        scratch_types=[
            pltpu.VMEM((BLK, CHUNK), jnp.int32),
            pltpu.VMEM((BLK, CHUNK), jnp.int32),
            pltpu.VMEM((CHUNK, D), jnp.float32),
            pltpu.VMEM((CHUNK, D), jnp.float32),
            pltpu.VMEM((CHUNK, CNTW), jnp.float32),
            pltpu.VMEM((CHUNK, CNTW), jnp.float32),
            pltpu.VMEM_SHARED((N, D), jnp.float32),
            pltpu.VMEM_SHARED((N, CNTW), jnp.float32),
            pltpu.SemaphoreType.DMA,
            pltpu.SemaphoreType.DMA,
        ],
    )(x1, x2, s1, d1, s2, d2)


def _tc_body(agg_ref, cnt_ref, w_ref, out_ref):
    w = w_ref[...]
    embs = []
    for c in range(2):
        a = agg_ref[c * N:(c + 1) * N, :]
        deg = cnt_ref[c * N:(c + 1) * N, 0:1]
        a = a / jnp.maximum(deg, 1.0)
        h = jnp.maximum(
            jax.lax.dot(a, w, preferred_element_type=jnp.float32), 0.0)
        embs.append(jnp.sum(h, axis=0, keepdims=True) / float(N))
    out_ref[...] = jnp.sum(embs[0] * embs[1]).reshape(1, 1)


@jax.jit
def _tc_finish(agg, cnt, W):
    return pl.pallas_call(
        _tc_body,
        out_shape=jax.ShapeDtypeStruct((1, 1), jnp.float32),
    )(agg, cnt, W)


def kernel(x1, x2, W, edge_index1, edge_index2):
    s1 = edge_index1[0].reshape(NCHUNKS, CHUNK)
    d1 = edge_index1[1].reshape(NCHUNKS, CHUNK)
    s2 = edge_index2[0].reshape(NCHUNKS, CHUNK)
    d2 = edge_index2[1].reshape(NCHUNKS, CHUNK)
    agg, cnt = _sc_aggregate(x1, x2, s1, d1, s2, d2)
    out = _tc_finish(agg, cnt, W)
    return out[0, 0]
